# baseline (speedup 1.0000x reference)
import functools

import jax
import jax.numpy as jnp
from jax import lax
from jax.experimental import pallas as pl
from jax.experimental.pallas import tpu as pltpu
from jax.experimental.pallas import tpu_sc as plsc

NUM_EMB = 1_000_000
DIM = 32
B_TOTAL = 16384 * 50

_NUM_CORES = 2
_NUM_SUBCORES = 16
_NUM_WORKERS = _NUM_CORES * _NUM_SUBCORES
_B_PER_W = B_TOTAL // _NUM_WORKERS

_SHARD = 32768
_N_BUCKET = 31
_N_FULL_PASS = 30
_LAST_ROWS = NUM_EMB - _N_FULL_PASS * _SHARD
_SH_PER = _SHARD // _NUM_SUBCORES
_C = 128
_CAP = _B_PER_W + _N_BUCKET * _C
_CROWS = _CAP // _C
_PART_CHUNK = 1280
_N_PART = _B_PER_W // _PART_CHUNK
_B_PAD = B_TOTAL + _C


def _body(idx_hbm, table_hbm, out_hbm,
          stage, segi, segp2, hraw, tmpv, tmpd,
          hist, cstart, ccur, rcv, sidx, rows,
          sem_st, sem_g, sem_o):
    cid = lax.axis_index("c")
    sid = lax.axis_index("s")
    wid = sid * _NUM_CORES + cid
    wbase = wid * _B_PER_W
    iota = lax.iota(jnp.int32, 16)
    zero16 = jnp.zeros((16,), jnp.int32)

    def vextract(ref32, j):
        g = plsc.load_gather(ref32, [jnp.full((16,), j, jnp.int32)])
        return jnp.sum(jnp.where(iota == 0, g, 0))

    def rank_of(v):
        sb, sperm = plsc.sort_key_val(v, iota)
        tmpv[...] = sb
        prev = plsc.load_gather(tmpv, [(iota + 15) & 15])
        seg_start = jnp.logical_or(iota == 0, sb != prev)
        m = plsc.cummax(jnp.where(seg_start, iota, 0))
        rank = iota - m
        nxt = plsc.load_gather(tmpv, [(iota + 1) & 15])
        is_end = jnp.logical_or(iota == 15, sb != nxt)
        return sb, sperm, rank, is_end

    hist[pl.ds(0, 16)] = zero16
    hist[pl.ds(16, 16)] = zero16

    def hist_chunk(c, carry):
        pltpu.sync_copy(
            idx_hbm.at[pl.ds(pl.multiple_of(wbase + c * _PART_CHUNK, _C),
                             _PART_CHUNK)], hraw)

        def hist_vreg(k, carry2):
            v = hraw[pl.ds(k * 16, 16)]
            sb, _, rank, is_end = rank_of(lax.shift_right_logical(v, 15))
            plsc.addupdate_scatter(hist, [sb], rank + 1, mask=is_end)
            return carry2

        lax.fori_loop(0, _PART_CHUNK // 16, hist_vreg, 0)
        return carry

    lax.fori_loop(0, _N_PART, hist_chunk, 0)

    h0 = hist[pl.ds(0, 16)]
    h1 = hist[pl.ds(16, 16)]
    rc0 = lax.shift_left(lax.shift_right_logical(h0 + 127, 7), 7)
    rc1 = lax.shift_left(lax.shift_right_logical(h1 + 127, 7), 7)
    cs0 = plsc.cumsum(rc0)
    cs1 = plsc.cumsum(rc1)
    tot0 = jnp.sum(rc0)
    s0 = cs0 - rc0
    s1 = cs1 - rc1 + tot0
    cstart[pl.ds(0, 16)] = s0
    cstart[pl.ds(16, 16)] = s1
    ccur[pl.ds(0, 16)] = s0
    ccur[pl.ds(16, 16)] = s1
    rcv[pl.ds(0, 16)] = rc0
    rcv[pl.ds(16, 16)] = rc1

    def pad_bucket(b, carry):
        rcb = vextract(rcv, b)
        csb = vextract(cstart, b)

        @pl.when(rcb > 0)
        def _():
            fillv = jnp.full((16,), 1, jnp.int32) * (b * _SHARD)
            off = csb + rcb - _C
            crow = lax.shift_right_logical(off, 7)
            for u in range(8):
                segi[pl.ds(off + u * 16, 16)] = fillv
                segp2[crow, pl.ds(u * 16, 16)] = B_TOTAL + iota + u * 16

        return carry

    lax.fori_loop(0, _N_BUCKET, pad_bucket, 0)

    def part_chunk(c, carry):
        cbase = wbase + c * _PART_CHUNK
        pltpu.sync_copy(idx_hbm.at[pl.ds(pl.multiple_of(cbase, _C),
                                         _PART_CHUNK)], hraw)

        def part_vreg(k, carry2):
            v = hraw[pl.ds(k * 16, 16)]
            sb, sperm, rank, is_end = rank_of(
                lax.shift_right_logical(v, 15))
            cur = plsc.load_gather(ccur, [sb])
            dsort = cur + rank
            plsc.store_scatter(ccur, [sb], dsort + 1, mask=is_end)
            plsc.store_scatter(tmpd, [sperm], dsort)
            dorig = tmpd[...]
            plsc.store_scatter(segi, [dorig], v)
            plsc.store_scatter(
                segp2,
                [lax.shift_right_logical(dorig, 7), dorig & (_C - 1)],
                cbase + k * 16 + iota)
            return carry2

        lax.fori_loop(0, _PART_CHUNK // 16, part_vreg, 0)
        return carry

    lax.fori_loop(0, _N_PART, part_chunk, 0)

    def run_pass(p, stage_rows_per_subcore):
        base_row = p * _SHARD
        pltpu.sync_copy(
            table_hbm.at[pl.ds(base_row + sid * stage_rows_per_subcore,
                               stage_rows_per_subcore)],
            stage.at[pl.ds(sid * stage_rows_per_subcore,
                           stage_rows_per_subcore)])
        plsc.subcore_barrier()
        cb = vextract(cstart, p)
        nck = lax.shift_right_logical(vextract(rcv, p), 7)
        crow0 = lax.shift_right_logical(cb, 7)

        def chunk(j, carry):
            for u in range(8):
                sidx[pl.ds(u * 16, 16)] = (
                    segi[pl.ds(cb + j * _C + u * 16, 16)] - base_row)
            pltpu.async_copy(stage.at[sidx], rows, sem_g).wait()
            pltpu.async_copy(rows, out_hbm.at[segp2.at[crow0 + j]],
                             sem_o).wait()
            return carry

        lax.fori_loop(0, nck, chunk, 0)
        plsc.subcore_barrier()

    def full_pass(p, carry):
        run_pass(p, _SH_PER)
        return carry

    lax.fori_loop(0, _N_FULL_PASS, full_pass, 0)
    run_pass(_N_FULL_PASS, _LAST_ROWS // _NUM_SUBCORES)


def kernel(token_ids, weight):
    idx = token_ids.reshape(-1).astype(jnp.int32)
    mesh = plsc.VectorSubcoreMesh(core_axis_name="c", subcore_axis_name="s")
    run = functools.partial(
        pl.kernel,
        mesh=mesh,
        out_type=jax.ShapeDtypeStruct((_B_PAD, DIM), jnp.float32),
        scratch_types=[
            pltpu.VMEM_SHARED((_SHARD, DIM), jnp.float32),
            pltpu.VMEM((_CAP,), jnp.int32),
            pltpu.VMEM((_CROWS, _C), jnp.int32),
            pltpu.VMEM((_PART_CHUNK,), jnp.int32),
            pltpu.VMEM((16,), jnp.int32),
            pltpu.VMEM((16,), jnp.int32),
            pltpu.VMEM((32,), jnp.int32),
            pltpu.VMEM((32,), jnp.int32),
            pltpu.VMEM((32,), jnp.int32),
            pltpu.VMEM((32,), jnp.int32),
            pltpu.VMEM((_C,), jnp.int32),
            pltpu.VMEM((_C, DIM), jnp.float32),
            pltpu.SemaphoreType.DMA,
            pltpu.SemaphoreType.DMA,
            pltpu.SemaphoreType.DMA,
        ],
        compiler_params=pltpu.CompilerParams(use_tc_tiling_on_sc=False,
                                             needs_layout_passes=False),
    )(_body)
    out = run(idx, weight)
    return out[:B_TOTAL].reshape(token_ids.shape[0], token_ids.shape[1], DIM)


# speedup vs baseline: 1.3570x; 1.3570x over previous
import functools

import jax
import jax.numpy as jnp
from jax import lax
from jax.experimental import pallas as pl
from jax.experimental.pallas import tpu as pltpu
from jax.experimental.pallas import tpu_sc as plsc

NUM_EMB = 1_000_000
DIM = 32
B_TOTAL = 16384 * 50

_NUM_CORES = 2
_NUM_SUBCORES = 16
_NUM_WORKERS = _NUM_CORES * _NUM_SUBCORES
_B_PER_W = B_TOTAL // _NUM_WORKERS
_CHUNK = 1600
_N_CHUNKS = _B_PER_W // _CHUNK
_N_STREAMS = 4
_SUB = _CHUNK // _N_STREAMS


def _gather_body(idx_hbm, table_hbm, out_hbm,
                 idx0, idx1, rows0, rows1, sem0, sem1):
    wid = lax.axis_index("s") * _NUM_CORES + lax.axis_index("c")
    base = wid * _B_PER_W

    def start_gather(c, idx_v, rows_v, sem):
        off = base + c * _CHUNK
        pltpu.sync_copy(idx_hbm.at[pl.ds(off, _CHUNK)], idx_v)
        for j in range(_N_STREAMS):
            pltpu.async_copy(
                table_hbm.at[idx_v.at[pl.ds(j * _SUB, _SUB)]],
                rows_v.at[pl.ds(j * _SUB, _SUB)],
                sem,
            )

    def wait_gather(idx_v, rows_v, sem):
        for j in range(_N_STREAMS):
            pltpu.make_async_copy(
                table_hbm.at[idx_v.at[pl.ds(j * _SUB, _SUB)]],
                rows_v.at[pl.ds(j * _SUB, _SUB)],
                sem,
            ).wait()

    def store(c, rows_v):
        pltpu.sync_copy(rows_v, out_hbm.at[pl.ds(base + c * _CHUNK, _CHUNK)])

    start_gather(0, idx0, rows0, sem0)

    def step(p, carry):
        g = 2 * p

        @pl.when(g + 1 < _N_CHUNKS)
        def _():
            start_gather(g + 1, idx1, rows1, sem1)

        wait_gather(idx0, rows0, sem0)
        store(g, rows0)

        @pl.when(g + 2 < _N_CHUNKS)
        def _():
            start_gather(g + 2, idx0, rows0, sem0)

        @pl.when(g + 1 < _N_CHUNKS)
        def _():
            wait_gather(idx1, rows1, sem1)
            store(g + 1, rows1)

        return carry

    lax.fori_loop(0, (_N_CHUNKS + 1) // 2, step, 0)


def kernel(token_ids, weight):
    idx = token_ids.reshape(-1).astype(jnp.int32)
    mesh = plsc.VectorSubcoreMesh(core_axis_name="c", subcore_axis_name="s")
    run = functools.partial(
        pl.kernel,
        mesh=mesh,
        out_type=jax.ShapeDtypeStruct((B_TOTAL, DIM), jnp.float32),
        scratch_types=[
            pltpu.VMEM((_CHUNK,), jnp.int32),
            pltpu.VMEM((_CHUNK,), jnp.int32),
            pltpu.VMEM((_CHUNK, DIM), jnp.float32),
            pltpu.VMEM((_CHUNK, DIM), jnp.float32),
            pltpu.SemaphoreType.DMA,
            pltpu.SemaphoreType.DMA,
        ],
        compiler_params=pltpu.CompilerParams(use_tc_tiling_on_sc=False),
    )(_gather_body)
    out = run(idx, weight)
    return out.reshape(token_ids.shape[0], token_ids.shape[1], DIM)
